# Initial kernel scaffold; baseline (speedup 1.0000x reference)
import jax
import jax.numpy as jnp
from jax.experimental import pallas as pl


def kernel(prods_only, user_features, product_info_features, product_name_features, edge_index, edge_attr, uW1, ub1, ug, ubeta, uW2, ub2, nW1, nb1, ng, nbeta, nW2, nb2, iW1, ib1, ig, ibeta, iW2, ib2, eW1, eb1, eg, ebeta, eW2, eb2, pW, pb, Wq, bq, Wk, bk, Wv, bv, We, Wskip, bskip):
    raise NotImplementedError("write your pallas kernel here")


# speedup vs baseline: 4.6091x; 4.6091x over previous
import jax
import jax.numpy as jnp
from jax import lax
from jax.experimental import pallas as pl
from jax.experimental.pallas import tpu as pltpu
from jax.experimental.pallas import tpu_sc as plsc

N_USERS = 25000
N_PRODUCTS = 25000
N_NODES = 50000
N_EDGES = 800000
D_NODE = 128
D_EDGE = 16
OUT_NODE = 64
HEADS = 4
HEAD_DIM = OUT_NODE // HEADS

NODE_BLK = 1000
EDGE_BLK = 4000

SC_CORES = 2
SC_TILES = 16
EB = 128
NBLK = N_EDGES // EB
ACC_ROWS = 25088
ACC_BLKS = ACC_ROWS // EB
ROW_W = 48
TBL_W = 32


def _dot_t(x, w):
    return jax.lax.dot_general(x, w, (((1,), (1,)), ((), ())),
                               preferred_element_type=jnp.float32)


def _ln_relu(h, g, beta):
    m = jnp.mean(h, axis=-1, keepdims=True)
    v = jnp.mean((h - m) ** 2, axis=-1, keepdims=True)
    h = (h - m) * jax.lax.rsqrt(v + 1e-5) * g + beta
    return jnp.maximum(h, 0.0)


def _split_heads(x_ref, x):
    x_ref[0] = x[:, :TBL_W]
    x_ref[1] = x[:, TBL_W:]


def _user_body(x_ref, w1, b1, g, beta, w2, b2, wq, bq, wk, bk, wv, bv, ws, bs,
               q_ref, k_ref, v_ref, skip_ref):
    x = x_ref[...]
    h = _ln_relu(_dot_t(x, w1[...]) + b1[...], g[...], beta[...])
    emb = _dot_t(h, w2[...]) + b2[...]
    _split_heads(q_ref, _dot_t(emb, wq[...]) + bq[...])
    _split_heads(k_ref, _dot_t(emb, wk[...]) + bk[...])
    _split_heads(v_ref, _dot_t(emb, wv[...]) + bv[...])
    skip_ref[...] = _dot_t(emb, ws[...]) + bs[...]


def _product_body(xi_ref, xn_ref,
                  iw1, ib1, ig, ibeta, iw2, ib2,
                  nw1, nb1, ng, nbeta, nw2, nb2,
                  pw, pb, wk, bk, wv, bv,
                  pemb_ref, k_ref, v_ref):
    hi = _ln_relu(_dot_t(xi_ref[...], iw1[...]) + ib1[...], ig[...], ibeta[...])
    info = _dot_t(hi, iw2[...]) + ib2[...]
    hn = _ln_relu(_dot_t(xn_ref[...], nw1[...]) + nb1[...], ng[...], nbeta[...])
    name = _dot_t(hn, nw2[...]) + nb2[...]
    cat = jnp.concatenate([name, info], axis=-1)
    pemb = _dot_t(cat, pw[...]) + pb[...]
    pemb_ref[...] = pemb
    _split_heads(k_ref, _dot_t(pemb, wk[...]) + bk[...])
    _split_heads(v_ref, _dot_t(pemb, wv[...]) + bv[...])


def _edge_body(x_ref, src_ref, off_ref, w1, b1, g, beta, w2, b2, we,
               e_ref, srcs_ref):
    h = _ln_relu(_dot_t(x_ref[...], w1[...]) + b1[...], g[...], beta[...])
    ee = _dot_t(h, w2[...]) + b2[...]
    _split_heads(e_ref, _dot_t(ee, we[...]))
    srcs_ref[...] = src_ref[...] + off_ref[0, 0]


def _max_body(dst_ref, out_ref):
    out_ref[...] = jnp.full((8, 128), jnp.max(dst_ref[...]), jnp.int32)


def _combine_body(t0_ref, t1_ref, skip_ref, out_ref):
    pieces = []
    for t_ref in (t0_ref, t1_ref):
        t = t_ref[...]
        for j in range(2):
            d = t[:, 2 * HEAD_DIM + j:2 * HEAD_DIM + j + 1]
            pieces.append(t[:, j * HEAD_DIM:(j + 1) * HEAD_DIM] / (d + 1e-16))
    out_ref[...] = jnp.concatenate(pieces, axis=-1) + skip_ref[...]


def _rep(shape):
    return pl.BlockSpec(shape, lambda i: (0,) * len(shape))


def _sc_body(q_hbm, k_hbm, v_hbm, e_hbm, src_hbm, dst_hbm, out_hbm,
             didx, didx2, sidx, qrows, krows, vrows, erows, cbuf, zbuf,
             acc, sem, sem2, sem3):
    c = lax.axis_index("c")
    s = lax.axis_index("s")

    def _zero_row(i, _):
        for j in range(ROW_W // 16):
            zbuf[i, pl.ds(j * 16, 16)] = jnp.zeros((16,), jnp.float32)
        return 0
    lax.fori_loop(0, EB, _zero_row, 0)

    def _zero_blk(t, _):
        blk = s + SC_TILES * t
        @pl.when(blk < ACC_BLKS)
        def _():
            pltpu.sync_copy(zbuf, acc.at[pl.ds(blk * EB, EB)])
        return 0
    lax.fori_loop(0, (ACC_BLKS + SC_TILES - 1) // SC_TILES, _zero_blk, 0)
    plsc.subcore_barrier()

    inv_sqrt = 1.0 / (HEAD_DIM ** 0.5)
    qoff = c * N_USERS
    soff = c * N_NODES
    eoff = c * N_EDGES

    dnums = lax.GatherDimensionNumbers(
        offset_dims=(), collapsed_slice_dims=(0,), start_index_map=(0,))

    def _bcast_last(x):
        c15 = jnp.full((16, 1), 15, jnp.int32)
        return lax.gather(x, c15, dnums, (1,),
                          mode=lax.GatherScatterMode.PROMISE_IN_BOUNDS)

    def _edge_blk(t, _):
        b = s + SC_TILES * t
        @pl.when(b < NBLK)
        def _():
            base = b * EB
            pltpu.sync_copy(dst_hbm.at[pl.ds(base, EB)], didx)
            pltpu.sync_copy(src_hbm.at[pl.ds(base, EB)], sidx)
            for j in range(EB // 16):
                sl = pl.ds(j * 16, 16)
                didx2[sl] = didx[sl] + qoff
                sidx[sl] = sidx[sl] + soff
            d1 = pltpu.async_copy(q_hbm.at[didx2], qrows, sem)
            d2 = pltpu.async_copy(k_hbm.at[sidx], krows, sem2)
            d3 = pltpu.async_copy(v_hbm.at[sidx], vrows, sem3)
            pltpu.sync_copy(e_hbm.at[pl.ds(eoff + base, EB)], erows)
            d1.wait()
            d2.wait()
            d3.wait()

            def _edge(i, _):
                iotl = lax.iota(jnp.int32, 16)
                avec = jnp.zeros((16,), jnp.float32)
                for h in range(2):
                    sl = pl.ds(h * HEAD_DIM, HEAD_DIM)
                    ev = erows[i, sl]
                    prod = qrows[i, sl] * (krows[i, sl] + ev)
                    af = _bcast_last(jnp.cumsum(prod)) * inv_sqrt
                    avec = jnp.where(iotl == h, af, avec)
                    wsp = jnp.exp(af)
                    cbuf[i, sl] = (vrows[i, sl] + ev) * wsp
                cbuf[i, pl.ds(2 * HEAD_DIM, 16)] = jnp.exp(avec)
                return 0
            lax.fori_loop(0, EB, _edge, 0)
            pltpu.sync_copy(cbuf, acc.at[didx], add=True)
        return 0
    lax.fori_loop(0, (NBLK + SC_TILES - 1) // SC_TILES, _edge_blk, 0)

    plsc.subcore_barrier()

    def _drain_blk(t, _):
        blk = s + SC_TILES * t
        @pl.when(blk < ACC_BLKS)
        def _():
            pltpu.sync_copy(acc.at[pl.ds(blk * EB, EB)],
                            out_hbm.at[c, pl.ds(blk * EB, EB)])
        return 0
    lax.fori_loop(0, (ACC_BLKS + SC_TILES - 1) // SC_TILES, _drain_blk, 0)


def _sc_message_passing(q2, k2, v2, e2, src_shift, dst):
    f32 = jnp.float32
    mesh = plsc.VectorSubcoreMesh(core_axis_name="c", subcore_axis_name="s",
                                  num_cores=SC_CORES, num_subcores=SC_TILES)
    run = pl.kernel(
        _sc_body,
        out_type=jax.ShapeDtypeStruct((SC_CORES, ACC_ROWS, ROW_W), f32),
        mesh=mesh,
        scratch_types=[
            pltpu.VMEM((EB,), jnp.int32),
            pltpu.VMEM((EB,), jnp.int32),
            pltpu.VMEM((EB,), jnp.int32),
            pltpu.VMEM((EB, TBL_W), f32),
            pltpu.VMEM((EB, TBL_W), f32),
            pltpu.VMEM((EB, TBL_W), f32),
            pltpu.VMEM((EB, TBL_W), f32),
            pltpu.VMEM((EB, ROW_W), f32),
            pltpu.VMEM((EB, ROW_W), f32),
            pltpu.VMEM_SHARED((ACC_ROWS, ROW_W), f32),
            pltpu.SemaphoreType.DMA,
            pltpu.SemaphoreType.DMA,
            pltpu.SemaphoreType.DMA,
        ],
        compiler_params=pltpu.CompilerParams(use_tc_tiling_on_sc=False,
                                             needs_layout_passes=False),
    )
    return run(q2, k2, v2, e2, src_shift, dst)


def kernel(prods_only, user_features, product_info_features, product_name_features,
           edge_index, edge_attr,
           uW1, ub1, ug, ubeta, uW2, ub2,
           nW1, nb1, ng, nbeta, nW2, nb2,
           iW1, ib1, ig, ibeta, iW2, ib2,
           eW1, eb1, eg, ebeta, eW2, eb2,
           pW, pb, Wq, bq, Wk, bk, Wv, bv, We, Wskip, bskip):
    f32 = jnp.float32
    nb = N_USERS // NODE_BLK

    def _heads_out():
        return pl.BlockSpec((SC_CORES, NODE_BLK, TBL_W), lambda i: (0, i, 0))

    q2u, k2u, v2u, skip_u = pl.pallas_call(
        _user_body,
        grid=(nb,),
        in_specs=[pl.BlockSpec((NODE_BLK, D_NODE), lambda i: (i, 0)),
                  _rep(uW1.shape), _rep(ub1.shape), _rep(ug.shape), _rep(ubeta.shape),
                  _rep(uW2.shape), _rep(ub2.shape),
                  _rep(Wq.shape), _rep(bq.shape), _rep(Wk.shape), _rep(bk.shape),
                  _rep(Wv.shape), _rep(bv.shape), _rep(Wskip.shape), _rep(bskip.shape)],
        out_specs=[_heads_out(), _heads_out(), _heads_out(),
                   pl.BlockSpec((NODE_BLK, OUT_NODE), lambda i: (i, 0))],
        out_shape=[jax.ShapeDtypeStruct((SC_CORES, N_USERS, TBL_W), f32)] * 3
                  + [jax.ShapeDtypeStruct((N_USERS, OUT_NODE), f32)],
    )(user_features, uW1, ub1, ug, ubeta, uW2, ub2,
      Wq, bq, Wk, bk, Wv, bv, Wskip, bskip)

    product_emb, k2p, v2p = pl.pallas_call(
        _product_body,
        grid=(nb,),
        in_specs=[pl.BlockSpec((NODE_BLK, D_NODE), lambda i: (i, 0)),
                  pl.BlockSpec((NODE_BLK, D_NODE), lambda i: (i, 0)),
                  _rep(iW1.shape), _rep(ib1.shape), _rep(ig.shape), _rep(ibeta.shape),
                  _rep(iW2.shape), _rep(ib2.shape),
                  _rep(nW1.shape), _rep(nb1.shape), _rep(ng.shape), _rep(nbeta.shape),
                  _rep(nW2.shape), _rep(nb2.shape),
                  _rep(pW.shape), _rep(pb.shape),
                  _rep(Wk.shape), _rep(bk.shape), _rep(Wv.shape), _rep(bv.shape)],
        out_specs=[pl.BlockSpec((NODE_BLK, OUT_NODE), lambda i: (i, 0)),
                   _heads_out(), _heads_out()],
        out_shape=[jax.ShapeDtypeStruct((N_PRODUCTS, OUT_NODE), f32)]
                  + [jax.ShapeDtypeStruct((SC_CORES, N_PRODUCTS, TBL_W), f32)] * 2,
    )(product_info_features, product_name_features,
      iW1, ib1, ig, ibeta, iW2, ib2,
      nW1, nb1, ng, nbeta, nW2, nb2,
      pW, pb, Wk, bk, Wv, bv)

    dst = edge_index[1]
    maxdst = pl.pallas_call(
        _max_body,
        in_specs=[pl.BlockSpec((800, 1000), lambda: (0, 0))],
        out_specs=pl.BlockSpec((8, 128), lambda: (0, 0)),
        out_shape=jax.ShapeDtypeStruct((8, 128), jnp.int32),
    )(dst.reshape(800, 1000))
    off_blk = maxdst + 1 + jnp.asarray(prods_only, jnp.int32)

    neb = N_EDGES // EDGE_BLK
    e2, src_shift3 = pl.pallas_call(
        _edge_body,
        grid=(neb,),
        in_specs=[pl.BlockSpec((EDGE_BLK, D_EDGE), lambda i: (i, 0)),
                  pl.BlockSpec((1, 1, EDGE_BLK), lambda i: (i, 0, 0)),
                  pl.BlockSpec((8, 128), lambda i: (0, 0)),
                  _rep(eW1.shape), _rep(eb1.shape), _rep(eg.shape), _rep(ebeta.shape),
                  _rep(eW2.shape), _rep(eb2.shape), _rep(We.shape)],
        out_specs=[pl.BlockSpec((SC_CORES, EDGE_BLK, TBL_W), lambda i: (0, i, 0)),
                   pl.BlockSpec((1, 1, EDGE_BLK), lambda i: (i, 0, 0))],
        out_shape=[jax.ShapeDtypeStruct((SC_CORES, N_EDGES, TBL_W), f32),
                   jax.ShapeDtypeStruct((neb, 1, EDGE_BLK), jnp.int32)],
    )(edge_attr, edge_index[0].reshape(neb, 1, EDGE_BLK), off_blk,
      eW1, eb1, eg, ebeta, eW2, eb2, We)

    k_flat = jnp.concatenate([k2u, k2p], axis=1).reshape(SC_CORES * N_NODES, TBL_W)
    v_flat = jnp.concatenate([v2u, v2p], axis=1).reshape(SC_CORES * N_NODES, TBL_W)
    q_flat = q2u.reshape(SC_CORES * N_USERS, TBL_W)
    e_flat = e2.reshape(SC_CORES * N_EDGES, TBL_W)

    out_sc = _sc_message_passing(q_flat, k_flat, v_flat, e_flat,
                                 src_shift3.reshape(N_EDGES), dst)

    out_u = pl.pallas_call(
        _combine_body,
        grid=(nb,),
        in_specs=[pl.BlockSpec((NODE_BLK, ROW_W), lambda i: (i, 0)),
                  pl.BlockSpec((NODE_BLK, ROW_W), lambda i: (i, 0)),
                  pl.BlockSpec((NODE_BLK, OUT_NODE), lambda i: (i, 0))],
        out_specs=pl.BlockSpec((NODE_BLK, OUT_NODE), lambda i: (i, 0)),
        out_shape=jax.ShapeDtypeStruct((N_USERS, OUT_NODE), f32),
    )(out_sc[0, :N_USERS], out_sc[1, :N_USERS], skip_u)

    return (out_u, product_emb)


# speedup vs baseline: 7.8915x; 1.7122x over previous
import jax
import jax.numpy as jnp
from jax import lax
from jax.experimental import pallas as pl
from jax.experimental.pallas import tpu as pltpu
from jax.experimental.pallas import tpu_sc as plsc

N_USERS = 25000
N_PRODUCTS = 25000
N_NODES = 50000
N_EDGES = 800000
D_NODE = 128
D_EDGE = 16
OUT_NODE = 64
HEADS = 4
HEAD_DIM = OUT_NODE // HEADS

NODE_BLK = 1000
EDGE_BLK = 4000

SC_CORES = 2
SC_TILES = 16
EB = 128
NBLK = N_EDGES // EB
ACC_ROWS = 25088
ACC_BLKS = ACC_ROWS // EB
ROW_W = 48
TBL_W = 32


def _dot_t(x, w):
    return jax.lax.dot_general(x, w, (((1,), (1,)), ((), ())),
                               preferred_element_type=jnp.float32)


def _ln_relu(h, g, beta):
    m = jnp.mean(h, axis=-1, keepdims=True)
    v = jnp.mean((h - m) ** 2, axis=-1, keepdims=True)
    h = (h - m) * jax.lax.rsqrt(v + 1e-5) * g + beta
    return jnp.maximum(h, 0.0)


def _split_heads(x_ref, x):
    x_ref[0] = x[:, :TBL_W]
    x_ref[1] = x[:, TBL_W:]


def _user_body(x_ref, w1, b1, g, beta, w2, b2, wq, bq, wk, bk, wv, bv, ws, bs,
               q_ref, k_ref, v_ref, skip_ref):
    x = x_ref[...]
    h = _ln_relu(_dot_t(x, w1[...]) + b1[...], g[...], beta[...])
    emb = _dot_t(h, w2[...]) + b2[...]
    _split_heads(q_ref, _dot_t(emb, wq[...]) + bq[...])
    _split_heads(k_ref, _dot_t(emb, wk[...]) + bk[...])
    _split_heads(v_ref, _dot_t(emb, wv[...]) + bv[...])
    skip_ref[...] = _dot_t(emb, ws[...]) + bs[...]


def _product_body(xi_ref, xn_ref,
                  iw1, ib1, ig, ibeta, iw2, ib2,
                  nw1, nb1, ng, nbeta, nw2, nb2,
                  pw, pb, wk, bk, wv, bv,
                  pemb_ref, k_ref, v_ref):
    hi = _ln_relu(_dot_t(xi_ref[...], iw1[...]) + ib1[...], ig[...], ibeta[...])
    info = _dot_t(hi, iw2[...]) + ib2[...]
    hn = _ln_relu(_dot_t(xn_ref[...], nw1[...]) + nb1[...], ng[...], nbeta[...])
    name = _dot_t(hn, nw2[...]) + nb2[...]
    cat = jnp.concatenate([name, info], axis=-1)
    pemb = _dot_t(cat, pw[...]) + pb[...]
    pemb_ref[...] = pemb
    _split_heads(k_ref, _dot_t(pemb, wk[...]) + bk[...])
    _split_heads(v_ref, _dot_t(pemb, wv[...]) + bv[...])


def _edge_body(x_ref, src_ref, off_ref, w1, b1, g, beta, w2, b2, we,
               e_ref, srcs_ref):
    h = _ln_relu(_dot_t(x_ref[...], w1[...]) + b1[...], g[...], beta[...])
    ee = _dot_t(h, w2[...]) + b2[...]
    _split_heads(e_ref, _dot_t(ee, we[...]))
    srcs_ref[...] = src_ref[...] + off_ref[0, 0]


def _max_body(dst_ref, out_ref):
    out_ref[...] = jnp.full((8, 128), jnp.max(dst_ref[...]), jnp.int32)


def _combine_body(t0_ref, t1_ref, skip_ref, out_ref):
    pieces = []
    for t_ref in (t0_ref, t1_ref):
        t = t_ref[...]
        for j in range(2):
            d = t[:, 2 * HEAD_DIM + j:2 * HEAD_DIM + j + 1]
            pieces.append(t[:, j * HEAD_DIM:(j + 1) * HEAD_DIM] / (d + 1e-16))
    out_ref[...] = jnp.concatenate(pieces, axis=-1) + skip_ref[...]


def _rep(shape):
    return pl.BlockSpec(shape, lambda i: (0,) * len(shape))


def _sc_body(q_hbm, k_hbm, v_hbm, e_hbm, src_hbm, dst_hbm, out_hbm,
             didx, didx2, sidx, qrows, krows, vrows, erows, cbuf, zbuf,
             acc, sem, sem2, sem3):
    c = lax.axis_index("c")
    s = lax.axis_index("s")

    def _zero_row(i, _):
        for j in range(ROW_W // 16):
            zbuf[i, pl.ds(j * 16, 16)] = jnp.zeros((16,), jnp.float32)
        return 0
    lax.fori_loop(0, EB, _zero_row, 0)

    def _zero_blk(t, _):
        blk = s + SC_TILES * t
        @pl.when(blk < ACC_BLKS)
        def _():
            pltpu.sync_copy(zbuf, acc.at[pl.ds(blk * EB, EB)])
        return 0
    lax.fori_loop(0, (ACC_BLKS + SC_TILES - 1) // SC_TILES, _zero_blk, 0)
    plsc.subcore_barrier()

    inv_sqrt = 1.0 / (HEAD_DIM ** 0.5)
    qoff = c * N_USERS
    soff = c * N_NODES
    eoff = c * N_EDGES

    dnums = lax.GatherDimensionNumbers(
        offset_dims=(), collapsed_slice_dims=(0,), start_index_map=(0,))

    def _bcast_last(x):
        c15 = jnp.full((16, 1), 15, jnp.int32)
        return lax.gather(x, c15, dnums, (1,),
                          mode=lax.GatherScatterMode.PROMISE_IN_BOUNDS)

    def _edge_blk(t, _):
        b = s + SC_TILES * t
        @pl.when(b < NBLK)
        def _():
            base = b * EB
            pltpu.sync_copy(dst_hbm.at[pl.ds(base, EB)], didx)
            pltpu.sync_copy(src_hbm.at[pl.ds(base, EB)], sidx)
            for j in range(EB // 16):
                sl = pl.ds(j * 16, 16)
                didx2[sl] = didx[sl] + qoff
                sidx[sl] = sidx[sl] + soff
            d1 = pltpu.async_copy(q_hbm.at[didx2], qrows, sem)
            d2 = pltpu.async_copy(k_hbm.at[sidx], krows, sem2)
            d3 = pltpu.async_copy(v_hbm.at[sidx], vrows, sem3)
            pltpu.sync_copy(e_hbm.at[pl.ds(eoff + base, EB)], erows)
            d1.wait()
            d2.wait()
            d3.wait()

            @plsc.parallel_loop(0, EB, unroll=4)
            def _edge(i):
                iotl = lax.iota(jnp.int32, 16)
                avec = jnp.zeros((16,), jnp.float32)
                for h in range(2):
                    sl = pl.ds(h * HEAD_DIM, HEAD_DIM)
                    ev = erows[i, sl]
                    prod = qrows[i, sl] * (krows[i, sl] + ev)
                    af = _bcast_last(jnp.cumsum(prod)) * inv_sqrt
                    avec = jnp.where(iotl == h, af, avec)
                    wsp = jnp.exp(af)
                    cbuf[i, sl] = (vrows[i, sl] + ev) * wsp
                cbuf[i, pl.ds(2 * HEAD_DIM, 16)] = jnp.exp(avec)
            pltpu.sync_copy(cbuf, acc.at[didx], add=True)
        return 0
    lax.fori_loop(0, (NBLK + SC_TILES - 1) // SC_TILES, _edge_blk, 0)

    plsc.subcore_barrier()

    def _drain_blk(t, _):
        blk = s + SC_TILES * t
        @pl.when(blk < ACC_BLKS)
        def _():
            pltpu.sync_copy(acc.at[pl.ds(blk * EB, EB)],
                            out_hbm.at[c, pl.ds(blk * EB, EB)])
        return 0
    lax.fori_loop(0, (ACC_BLKS + SC_TILES - 1) // SC_TILES, _drain_blk, 0)


def _sc_message_passing(q2, k2, v2, e2, src_shift, dst):
    f32 = jnp.float32
    mesh = plsc.VectorSubcoreMesh(core_axis_name="c", subcore_axis_name="s",
                                  num_cores=SC_CORES, num_subcores=SC_TILES)
    run = pl.kernel(
        _sc_body,
        out_type=jax.ShapeDtypeStruct((SC_CORES, ACC_ROWS, ROW_W), f32),
        mesh=mesh,
        scratch_types=[
            pltpu.VMEM((EB,), jnp.int32),
            pltpu.VMEM((EB,), jnp.int32),
            pltpu.VMEM((EB,), jnp.int32),
            pltpu.VMEM((EB, TBL_W), f32),
            pltpu.VMEM((EB, TBL_W), f32),
            pltpu.VMEM((EB, TBL_W), f32),
            pltpu.VMEM((EB, TBL_W), f32),
            pltpu.VMEM((EB, ROW_W), f32),
            pltpu.VMEM((EB, ROW_W), f32),
            pltpu.VMEM_SHARED((ACC_ROWS, ROW_W), f32),
            pltpu.SemaphoreType.DMA,
            pltpu.SemaphoreType.DMA,
            pltpu.SemaphoreType.DMA,
        ],
        compiler_params=pltpu.CompilerParams(use_tc_tiling_on_sc=False,
                                             needs_layout_passes=False),
    )
    return run(q2, k2, v2, e2, src_shift, dst)


def kernel(prods_only, user_features, product_info_features, product_name_features,
           edge_index, edge_attr,
           uW1, ub1, ug, ubeta, uW2, ub2,
           nW1, nb1, ng, nbeta, nW2, nb2,
           iW1, ib1, ig, ibeta, iW2, ib2,
           eW1, eb1, eg, ebeta, eW2, eb2,
           pW, pb, Wq, bq, Wk, bk, Wv, bv, We, Wskip, bskip):
    f32 = jnp.float32
    nb = N_USERS // NODE_BLK

    def _heads_out():
        return pl.BlockSpec((SC_CORES, NODE_BLK, TBL_W), lambda i: (0, i, 0))

    q2u, k2u, v2u, skip_u = pl.pallas_call(
        _user_body,
        grid=(nb,),
        in_specs=[pl.BlockSpec((NODE_BLK, D_NODE), lambda i: (i, 0)),
                  _rep(uW1.shape), _rep(ub1.shape), _rep(ug.shape), _rep(ubeta.shape),
                  _rep(uW2.shape), _rep(ub2.shape),
                  _rep(Wq.shape), _rep(bq.shape), _rep(Wk.shape), _rep(bk.shape),
                  _rep(Wv.shape), _rep(bv.shape), _rep(Wskip.shape), _rep(bskip.shape)],
        out_specs=[_heads_out(), _heads_out(), _heads_out(),
                   pl.BlockSpec((NODE_BLK, OUT_NODE), lambda i: (i, 0))],
        out_shape=[jax.ShapeDtypeStruct((SC_CORES, N_USERS, TBL_W), f32)] * 3
                  + [jax.ShapeDtypeStruct((N_USERS, OUT_NODE), f32)],
    )(user_features, uW1, ub1, ug, ubeta, uW2, ub2,
      Wq, bq, Wk, bk, Wv, bv, Wskip, bskip)

    product_emb, k2p, v2p = pl.pallas_call(
        _product_body,
        grid=(nb,),
        in_specs=[pl.BlockSpec((NODE_BLK, D_NODE), lambda i: (i, 0)),
                  pl.BlockSpec((NODE_BLK, D_NODE), lambda i: (i, 0)),
                  _rep(iW1.shape), _rep(ib1.shape), _rep(ig.shape), _rep(ibeta.shape),
                  _rep(iW2.shape), _rep(ib2.shape),
                  _rep(nW1.shape), _rep(nb1.shape), _rep(ng.shape), _rep(nbeta.shape),
                  _rep(nW2.shape), _rep(nb2.shape),
                  _rep(pW.shape), _rep(pb.shape),
                  _rep(Wk.shape), _rep(bk.shape), _rep(Wv.shape), _rep(bv.shape)],
        out_specs=[pl.BlockSpec((NODE_BLK, OUT_NODE), lambda i: (i, 0)),
                   _heads_out(), _heads_out()],
        out_shape=[jax.ShapeDtypeStruct((N_PRODUCTS, OUT_NODE), f32)]
                  + [jax.ShapeDtypeStruct((SC_CORES, N_PRODUCTS, TBL_W), f32)] * 2,
    )(product_info_features, product_name_features,
      iW1, ib1, ig, ibeta, iW2, ib2,
      nW1, nb1, ng, nbeta, nW2, nb2,
      pW, pb, Wk, bk, Wv, bv)

    dst = edge_index[1]
    maxdst = pl.pallas_call(
        _max_body,
        in_specs=[pl.BlockSpec((800, 1000), lambda: (0, 0))],
        out_specs=pl.BlockSpec((8, 128), lambda: (0, 0)),
        out_shape=jax.ShapeDtypeStruct((8, 128), jnp.int32),
    )(dst.reshape(800, 1000))
    off_blk = maxdst + 1 + jnp.asarray(prods_only, jnp.int32)

    neb = N_EDGES // EDGE_BLK
    e2, src_shift3 = pl.pallas_call(
        _edge_body,
        grid=(neb,),
        in_specs=[pl.BlockSpec((EDGE_BLK, D_EDGE), lambda i: (i, 0)),
                  pl.BlockSpec((1, 1, EDGE_BLK), lambda i: (i, 0, 0)),
                  pl.BlockSpec((8, 128), lambda i: (0, 0)),
                  _rep(eW1.shape), _rep(eb1.shape), _rep(eg.shape), _rep(ebeta.shape),
                  _rep(eW2.shape), _rep(eb2.shape), _rep(We.shape)],
        out_specs=[pl.BlockSpec((SC_CORES, EDGE_BLK, TBL_W), lambda i: (0, i, 0)),
                   pl.BlockSpec((1, 1, EDGE_BLK), lambda i: (i, 0, 0))],
        out_shape=[jax.ShapeDtypeStruct((SC_CORES, N_EDGES, TBL_W), f32),
                   jax.ShapeDtypeStruct((neb, 1, EDGE_BLK), jnp.int32)],
    )(edge_attr, edge_index[0].reshape(neb, 1, EDGE_BLK), off_blk,
      eW1, eb1, eg, ebeta, eW2, eb2, We)

    k_flat = jnp.concatenate([k2u, k2p], axis=1).reshape(SC_CORES * N_NODES, TBL_W)
    v_flat = jnp.concatenate([v2u, v2p], axis=1).reshape(SC_CORES * N_NODES, TBL_W)
    q_flat = q2u.reshape(SC_CORES * N_USERS, TBL_W)
    e_flat = e2.reshape(SC_CORES * N_EDGES, TBL_W)

    out_sc = _sc_message_passing(q_flat, k_flat, v_flat, e_flat,
                                 src_shift3.reshape(N_EDGES), dst)

    out_u = pl.pallas_call(
        _combine_body,
        grid=(nb,),
        in_specs=[pl.BlockSpec((NODE_BLK, ROW_W), lambda i: (i, 0)),
                  pl.BlockSpec((NODE_BLK, ROW_W), lambda i: (i, 0)),
                  pl.BlockSpec((NODE_BLK, OUT_NODE), lambda i: (i, 0))],
        out_specs=pl.BlockSpec((NODE_BLK, OUT_NODE), lambda i: (i, 0)),
        out_shape=jax.ShapeDtypeStruct((N_USERS, OUT_NODE), f32),
    )(out_sc[0, :N_USERS], out_sc[1, :N_USERS], skip_u)

    return (out_u, product_emb)


# speedup vs baseline: 10.8656x; 1.3769x over previous
import jax
import jax.numpy as jnp
from jax import lax
from jax.experimental import pallas as pl
from jax.experimental.pallas import tpu as pltpu
from jax.experimental.pallas import tpu_sc as plsc

N_USERS = 25000
N_PRODUCTS = 25000
N_NODES = 50000
N_EDGES = 800000
D_NODE = 128
D_EDGE = 16
OUT_NODE = 64
HEADS = 4
HEAD_DIM = OUT_NODE // HEADS

NODE_BLK = 1000
EDGE_BLK = 4000

SC_CORES = 2
SC_TILES = 16
EB = 128
NBLK = N_EDGES // EB
ACC_ROWS = 25088
ACC_BLKS = ACC_ROWS // EB
ROW_W = 48
TBL_W = 32


def _dot_t(x, w):
    return jax.lax.dot_general(x, w, (((1,), (1,)), ((), ())),
                               preferred_element_type=jnp.float32)


def _ln_relu(h, g, beta):
    m = jnp.mean(h, axis=-1, keepdims=True)
    v = jnp.mean((h - m) ** 2, axis=-1, keepdims=True)
    h = (h - m) * jax.lax.rsqrt(v + 1e-5) * g + beta
    return jnp.maximum(h, 0.0)


def _split_heads(x_ref, x):
    x_ref[0] = x[:, :TBL_W]
    x_ref[1] = x[:, TBL_W:]


def _user_body(x_ref, w1, b1, g, beta, w2, b2, wq, bq, wk, bk, wv, bv, ws, bs,
               q_ref, k_ref, v_ref, skip_ref):
    x = x_ref[...]
    h = _ln_relu(_dot_t(x, w1[...]) + b1[...], g[...], beta[...])
    emb = _dot_t(h, w2[...]) + b2[...]
    _split_heads(q_ref, (_dot_t(emb, wq[...]) + bq[...]) * 0.25)
    _split_heads(k_ref, _dot_t(emb, wk[...]) + bk[...])
    _split_heads(v_ref, _dot_t(emb, wv[...]) + bv[...])
    skip_ref[...] = _dot_t(emb, ws[...]) + bs[...]


def _product_body(xi_ref, xn_ref,
                  iw1, ib1, ig, ibeta, iw2, ib2,
                  nw1, nb1, ng, nbeta, nw2, nb2,
                  pw, pb, wk, bk, wv, bv,
                  pemb_ref, k_ref, v_ref):
    hi = _ln_relu(_dot_t(xi_ref[...], iw1[...]) + ib1[...], ig[...], ibeta[...])
    info = _dot_t(hi, iw2[...]) + ib2[...]
    hn = _ln_relu(_dot_t(xn_ref[...], nw1[...]) + nb1[...], ng[...], nbeta[...])
    name = _dot_t(hn, nw2[...]) + nb2[...]
    cat = jnp.concatenate([name, info], axis=-1)
    pemb = _dot_t(cat, pw[...]) + pb[...]
    pemb_ref[...] = pemb
    _split_heads(k_ref, _dot_t(pemb, wk[...]) + bk[...])
    _split_heads(v_ref, _dot_t(pemb, wv[...]) + bv[...])


def _edge_body(x_ref, src_ref, off_ref, w1, b1, g, beta, w2, b2, we,
               e_ref, srcs_ref):
    h = _ln_relu(_dot_t(x_ref[...], w1[...]) + b1[...], g[...], beta[...])
    ee = _dot_t(h, w2[...]) + b2[...]
    ew = _dot_t(ee, we[...])
    for ci in range(2):
        x4 = ew[:, ci * TBL_W:(ci + 1) * TBL_W].reshape(EDGE_BLK // 4, 4, TBL_W)
        e_ref[ci] = jnp.concatenate([x4[:, 0], x4[:, 1], x4[:, 2], x4[:, 3]],
                                    axis=-1)
    srcs_ref[...] = src_ref[...] + off_ref[0, 0]


def _max_body(dst_ref, out_ref):
    out_ref[...] = jnp.full((8, 128), jnp.max(dst_ref[...]), jnp.int32)


def _combine_body(t0_ref, t1_ref, skip_ref, out_ref):
    pieces = []
    for t_ref in (t0_ref, t1_ref):
        t = t_ref[...]
        for j in range(2):
            d = t[:, 2 * HEAD_DIM + j:2 * HEAD_DIM + j + 1]
            pieces.append(t[:, j * HEAD_DIM:(j + 1) * HEAD_DIM] / (d + 1e-16))
    out_ref[...] = jnp.concatenate(pieces, axis=-1) + skip_ref[...]


def _rep(shape):
    return pl.BlockSpec(shape, lambda i: (0,) * len(shape))


NT = (NBLK + SC_TILES - 1) // SC_TILES
NU = (NT + 1) // 2
EB4 = EB // 4


def _sc_body(q_hbm, k_hbm, v_hbm, e_hbm, src_hbm, dst_hbm, out_hbm,
             didx0, didx20, sidx0, q0, k0, v0, e0,
             didx1, didx21, sidx1, q1, k1, v1, e1,
             cbuf, zbuf, acc,
             sq0, sk0, sv0, se0, sq1, sk1, sv1, se1):
    c = lax.axis_index("c")
    s = lax.axis_index("s")

    def _zero_row(i, _):
        for j in range(ROW_W // 16):
            zbuf[i, pl.ds(j * 16, 16)] = jnp.zeros((16,), jnp.float32)
        return 0
    lax.fori_loop(0, EB, _zero_row, 0)

    def _zero_blk(t, _):
        blk = s + SC_TILES * t
        @pl.when(blk < ACC_BLKS)
        def _():
            pltpu.sync_copy(zbuf, acc.at[pl.ds(blk * EB, EB)])
        return 0
    lax.fori_loop(0, (ACC_BLKS + SC_TILES - 1) // SC_TILES, _zero_blk, 0)
    plsc.subcore_barrier()

    qoff = c * N_USERS
    soff = c * N_NODES
    eoff4 = c * (N_EDGES // 4)

    bufs = ((didx0, didx20, sidx0, q0, k0, v0, e0, sq0, sk0, sv0, se0),
            (didx1, didx21, sidx1, q1, k1, v1, e1, sq1, sk1, sv1, se1))

    def _fire(p, t):
        di, di2, si, qr, kr, vr, er, q_s, k_s, v_s, e_s = bufs[p]
        b = s + SC_TILES * t
        @pl.when(b < NBLK)
        def _():
            base = b * EB
            pltpu.sync_copy(dst_hbm.at[pl.ds(base, EB)], di)
            pltpu.sync_copy(src_hbm.at[pl.ds(base, EB)], si)
            for j in range(EB // 16):
                sl = pl.ds(j * 16, 16)
                di2[sl] = di[sl] + qoff
                si[sl] = si[sl] + soff
            pltpu.async_copy(q_hbm.at[di2], qr, q_s)
            pltpu.async_copy(k_hbm.at[si], kr, k_s)
            pltpu.async_copy(v_hbm.at[si], vr, v_s)
            pltpu.async_copy(e_hbm.at[pl.ds(eoff4 + b * EB4, EB4)], er, e_s)

    dnums = lax.GatherDimensionNumbers(
        offset_dims=(), collapsed_slice_dims=(0,), start_index_map=(0,))

    def _bcast_last(x):
        c15 = jnp.full((16, 1), 15, jnp.int32)
        return lax.gather(x, c15, dnums, (1,),
                          mode=lax.GatherScatterMode.PROMISE_IN_BOUNDS)

    def _consume(p, t):
        di, di2, si, qr, kr, vr, er, q_s, k_s, v_s, e_s = bufs[p]
        b = s + SC_TILES * t
        @pl.when(b < NBLK)
        def _():
            pltpu.make_async_copy(q_hbm.at[di2], qr, q_s).wait()
            pltpu.make_async_copy(k_hbm.at[si], kr, k_s).wait()
            pltpu.make_async_copy(v_hbm.at[si], vr, v_s).wait()
            pltpu.make_async_copy(e_hbm.at[pl.ds(0, EB4)], er, e_s).wait()

            @plsc.parallel_loop(0, EB, unroll=4)
            def _edge(i):
                iotl = lax.iota(jnp.int32, 16)
                r = i // 4
                sub = (i - 4 * r) * TBL_W
                avec = jnp.zeros((16,), jnp.float32)
                for h in range(2):
                    sl = pl.ds(h * HEAD_DIM, HEAD_DIM)
                    ev = er[r, pl.ds(sub + h * HEAD_DIM, HEAD_DIM)]
                    prod = qr[i, sl] * (kr[i, sl] + ev)
                    af = _bcast_last(jnp.cumsum(prod))
                    avec = jnp.where(iotl == h, af, avec)
                    wsp = jnp.exp(af)
                    cbuf[i, sl] = (vr[i, sl] + ev) * wsp
                cbuf[i, pl.ds(2 * HEAD_DIM, 16)] = jnp.exp(avec)
            pltpu.sync_copy(cbuf, acc.at[di], add=True)

    _fire(0, 0)

    def _step(u, _):
        _fire(1, 2 * u + 1)
        _consume(0, 2 * u)
        _fire(0, 2 * u + 2)
        _consume(1, 2 * u + 1)
        return 0
    lax.fori_loop(0, NU, _step, 0)

    plsc.subcore_barrier()

    def _drain_blk(t, _):
        blk = s + SC_TILES * t
        @pl.when(blk < ACC_BLKS)
        def _():
            pltpu.sync_copy(acc.at[pl.ds(blk * EB, EB)],
                            out_hbm.at[c, pl.ds(blk * EB, EB)])
        return 0
    lax.fori_loop(0, (ACC_BLKS + SC_TILES - 1) // SC_TILES, _drain_blk, 0)


def _sc_message_passing(q2, k2, v2, e2, src_shift, dst):
    f32 = jnp.float32
    i32 = jnp.int32
    mesh = plsc.VectorSubcoreMesh(core_axis_name="c", subcore_axis_name="s",
                                  num_cores=SC_CORES, num_subcores=SC_TILES)
    bufset = [pltpu.VMEM((EB,), i32), pltpu.VMEM((EB,), i32),
              pltpu.VMEM((EB,), i32),
              pltpu.VMEM((EB, TBL_W), f32), pltpu.VMEM((EB, TBL_W), f32),
              pltpu.VMEM((EB, TBL_W), f32), pltpu.VMEM((EB4, 128), f32)]
    run = pl.kernel(
        _sc_body,
        out_type=jax.ShapeDtypeStruct((SC_CORES, ACC_ROWS, ROW_W), f32),
        mesh=mesh,
        scratch_types=bufset + bufset + [
            pltpu.VMEM((EB, ROW_W), f32),
            pltpu.VMEM((EB, ROW_W), f32),
            pltpu.VMEM_SHARED((ACC_ROWS, ROW_W), f32),
        ] + [pltpu.SemaphoreType.DMA] * 8,
        compiler_params=pltpu.CompilerParams(use_tc_tiling_on_sc=False,
                                             needs_layout_passes=False),
    )
    return run(q2, k2, v2, e2, src_shift, dst)


def kernel(prods_only, user_features, product_info_features, product_name_features,
           edge_index, edge_attr,
           uW1, ub1, ug, ubeta, uW2, ub2,
           nW1, nb1, ng, nbeta, nW2, nb2,
           iW1, ib1, ig, ibeta, iW2, ib2,
           eW1, eb1, eg, ebeta, eW2, eb2,
           pW, pb, Wq, bq, Wk, bk, Wv, bv, We, Wskip, bskip):
    f32 = jnp.float32
    nb = N_USERS // NODE_BLK

    def _heads_out():
        return pl.BlockSpec((SC_CORES, NODE_BLK, TBL_W), lambda i: (0, i, 0))

    q2u, k2u, v2u, skip_u = pl.pallas_call(
        _user_body,
        grid=(nb,),
        in_specs=[pl.BlockSpec((NODE_BLK, D_NODE), lambda i: (i, 0)),
                  _rep(uW1.shape), _rep(ub1.shape), _rep(ug.shape), _rep(ubeta.shape),
                  _rep(uW2.shape), _rep(ub2.shape),
                  _rep(Wq.shape), _rep(bq.shape), _rep(Wk.shape), _rep(bk.shape),
                  _rep(Wv.shape), _rep(bv.shape), _rep(Wskip.shape), _rep(bskip.shape)],
        out_specs=[_heads_out(), _heads_out(), _heads_out(),
                   pl.BlockSpec((NODE_BLK, OUT_NODE), lambda i: (i, 0))],
        out_shape=[jax.ShapeDtypeStruct((SC_CORES, N_USERS, TBL_W), f32)] * 3
                  + [jax.ShapeDtypeStruct((N_USERS, OUT_NODE), f32)],
    )(user_features, uW1, ub1, ug, ubeta, uW2, ub2,
      Wq, bq, Wk, bk, Wv, bv, Wskip, bskip)

    product_emb, k2p, v2p = pl.pallas_call(
        _product_body,
        grid=(nb,),
        in_specs=[pl.BlockSpec((NODE_BLK, D_NODE), lambda i: (i, 0)),
                  pl.BlockSpec((NODE_BLK, D_NODE), lambda i: (i, 0)),
                  _rep(iW1.shape), _rep(ib1.shape), _rep(ig.shape), _rep(ibeta.shape),
                  _rep(iW2.shape), _rep(ib2.shape),
                  _rep(nW1.shape), _rep(nb1.shape), _rep(ng.shape), _rep(nbeta.shape),
                  _rep(nW2.shape), _rep(nb2.shape),
                  _rep(pW.shape), _rep(pb.shape),
                  _rep(Wk.shape), _rep(bk.shape), _rep(Wv.shape), _rep(bv.shape)],
        out_specs=[pl.BlockSpec((NODE_BLK, OUT_NODE), lambda i: (i, 0)),
                   _heads_out(), _heads_out()],
        out_shape=[jax.ShapeDtypeStruct((N_PRODUCTS, OUT_NODE), f32)]
                  + [jax.ShapeDtypeStruct((SC_CORES, N_PRODUCTS, TBL_W), f32)] * 2,
    )(product_info_features, product_name_features,
      iW1, ib1, ig, ibeta, iW2, ib2,
      nW1, nb1, ng, nbeta, nW2, nb2,
      pW, pb, Wk, bk, Wv, bv)

    dst = edge_index[1]
    maxdst = pl.pallas_call(
        _max_body,
        in_specs=[pl.BlockSpec((800, 1000), lambda: (0, 0))],
        out_specs=pl.BlockSpec((8, 128), lambda: (0, 0)),
        out_shape=jax.ShapeDtypeStruct((8, 128), jnp.int32),
    )(dst.reshape(800, 1000))
    off_blk = maxdst + 1 + jnp.asarray(prods_only, jnp.int32)

    neb = N_EDGES // EDGE_BLK
    e2, src_shift3 = pl.pallas_call(
        _edge_body,
        grid=(neb,),
        in_specs=[pl.BlockSpec((EDGE_BLK, D_EDGE), lambda i: (i, 0)),
                  pl.BlockSpec((1, 1, EDGE_BLK), lambda i: (i, 0, 0)),
                  pl.BlockSpec((8, 128), lambda i: (0, 0)),
                  _rep(eW1.shape), _rep(eb1.shape), _rep(eg.shape), _rep(ebeta.shape),
                  _rep(eW2.shape), _rep(eb2.shape), _rep(We.shape)],
        out_specs=[pl.BlockSpec((SC_CORES, EDGE_BLK // 4, 128), lambda i: (0, i, 0)),
                   pl.BlockSpec((1, 1, EDGE_BLK), lambda i: (i, 0, 0))],
        out_shape=[jax.ShapeDtypeStruct((SC_CORES, N_EDGES // 4, 128), f32),
                   jax.ShapeDtypeStruct((neb, 1, EDGE_BLK), jnp.int32)],
    )(edge_attr, edge_index[0].reshape(neb, 1, EDGE_BLK), off_blk,
      eW1, eb1, eg, ebeta, eW2, eb2, We)

    k_flat = jnp.concatenate([k2u, k2p], axis=1).reshape(SC_CORES * N_NODES, TBL_W)
    v_flat = jnp.concatenate([v2u, v2p], axis=1).reshape(SC_CORES * N_NODES, TBL_W)
    q_flat = q2u.reshape(SC_CORES * N_USERS, TBL_W)
    e_flat = e2.reshape(SC_CORES * N_EDGES // 4, 128)

    out_sc = _sc_message_passing(q_flat, k_flat, v_flat, e_flat,
                                 src_shift3.reshape(N_EDGES), dst)

    out_u = pl.pallas_call(
        _combine_body,
        grid=(nb,),
        in_specs=[pl.BlockSpec((NODE_BLK, ROW_W), lambda i: (i, 0)),
                  pl.BlockSpec((NODE_BLK, ROW_W), lambda i: (i, 0)),
                  pl.BlockSpec((NODE_BLK, OUT_NODE), lambda i: (i, 0))],
        out_specs=pl.BlockSpec((NODE_BLK, OUT_NODE), lambda i: (i, 0)),
        out_shape=jax.ShapeDtypeStruct((N_USERS, OUT_NODE), f32),
    )(out_sc[0, :N_USERS], out_sc[1, :N_USERS], skip_u)

    return (out_u, product_emb)


# speedup vs baseline: 10.9392x; 1.0068x over previous
import jax
import jax.numpy as jnp
from jax import lax
from jax.experimental import pallas as pl
from jax.experimental.pallas import tpu as pltpu
from jax.experimental.pallas import tpu_sc as plsc

N_USERS = 25000
N_PRODUCTS = 25000
N_NODES = 50000
N_EDGES = 800000
D_NODE = 128
D_EDGE = 16
OUT_NODE = 64
HEADS = 4
HEAD_DIM = OUT_NODE // HEADS

NODE_BLK = 1000
EDGE_BLK = 8000

SC_CORES = 2
SC_TILES = 16
EB = 128
NBLK = N_EDGES // EB
ACC_ROWS = 25088
ACC_BLKS = ACC_ROWS // EB
ROW_W = 48
TBL_W = 32


def _dot_t(x, w):
    return jax.lax.dot_general(x, w, (((1,), (1,)), ((), ())),
                               preferred_element_type=jnp.float32)


def _ln_relu(h, g, beta):
    m = jnp.mean(h, axis=-1, keepdims=True)
    v = jnp.mean((h - m) ** 2, axis=-1, keepdims=True)
    h = (h - m) * jax.lax.rsqrt(v + 1e-5) * g + beta
    return jnp.maximum(h, 0.0)


def _split_heads(x_ref, x):
    x_ref[0] = x[:, :TBL_W]
    x_ref[1] = x[:, TBL_W:]


def _user_body(x_ref, w1, b1, g, beta, w2, b2, wq, bq, wk, bk, wv, bv, ws, bs,
               q_ref, k_ref, v_ref, skip_ref):
    x = x_ref[...]
    h = _ln_relu(_dot_t(x, w1[...]) + b1[...], g[...], beta[...])
    emb = _dot_t(h, w2[...]) + b2[...]
    _split_heads(q_ref, (_dot_t(emb, wq[...]) + bq[...]) * 0.25)
    _split_heads(k_ref, _dot_t(emb, wk[...]) + bk[...])
    _split_heads(v_ref, _dot_t(emb, wv[...]) + bv[...])
    skip_ref[...] = _dot_t(emb, ws[...]) + bs[...]


def _product_body(xi_ref, xn_ref,
                  iw1, ib1, ig, ibeta, iw2, ib2,
                  nw1, nb1, ng, nbeta, nw2, nb2,
                  pw, pb, wk, bk, wv, bv,
                  pemb_ref, k_ref, v_ref):
    hi = _ln_relu(_dot_t(xi_ref[...], iw1[...]) + ib1[...], ig[...], ibeta[...])
    info = _dot_t(hi, iw2[...]) + ib2[...]
    hn = _ln_relu(_dot_t(xn_ref[...], nw1[...]) + nb1[...], ng[...], nbeta[...])
    name = _dot_t(hn, nw2[...]) + nb2[...]
    cat = jnp.concatenate([name, info], axis=-1)
    pemb = _dot_t(cat, pw[...]) + pb[...]
    pemb_ref[...] = pemb
    _split_heads(k_ref, _dot_t(pemb, wk[...]) + bk[...])
    _split_heads(v_ref, _dot_t(pemb, wv[...]) + bv[...])


def _edge_body(x_ref, src_ref, off_ref, w1, b1, g, beta, w2, b2, we,
               e_ref, srcs_ref):
    h = _ln_relu(_dot_t(x_ref[...], w1[...]) + b1[...], g[...], beta[...])
    w2e = jax.lax.dot_general(we[...], w2[...], (((1,), (0,)), ((), ())),
                              preferred_element_type=jnp.float32)
    b2e = jax.lax.dot_general(we[...], b2[...], (((1,), (0,)), ((), ())),
                              preferred_element_type=jnp.float32)
    ew = _dot_t(h, w2e) + b2e
    for ci in range(2):
        x4 = ew[:, ci * TBL_W:(ci + 1) * TBL_W].reshape(EDGE_BLK // 4, 4, TBL_W)
        e_ref[ci] = jnp.concatenate([x4[:, 0], x4[:, 1], x4[:, 2], x4[:, 3]],
                                    axis=-1)
    srcs_ref[...] = src_ref[...] + off_ref[0, 0]


def _max_body(dst_ref, out_ref):
    out_ref[...] = jnp.full((8, 128), jnp.max(dst_ref[...]), jnp.int32)


def _combine_body(t0_ref, t1_ref, skip_ref, out_ref):
    pieces = []
    for t_ref in (t0_ref, t1_ref):
        t = t_ref[...]
        for j in range(2):
            d = t[:, 2 * HEAD_DIM + j:2 * HEAD_DIM + j + 1]
            pieces.append(t[:, j * HEAD_DIM:(j + 1) * HEAD_DIM] / (d + 1e-16))
    out_ref[...] = jnp.concatenate(pieces, axis=-1) + skip_ref[...]


def _rep(shape):
    return pl.BlockSpec(shape, lambda i: (0,) * len(shape))


NT = (NBLK + SC_TILES - 1) // SC_TILES
NU = (NT + 1) // 2
EB4 = EB // 4


def _sc_body(q_hbm, k_hbm, v_hbm, e_hbm, src_hbm, dst_hbm, out_hbm,
             didx0, didx20, sidx0, q0, k0, v0, e0,
             didx1, didx21, sidx1, q1, k1, v1, e1,
             cbuf, zbuf, acc,
             sq0, sk0, sv0, se0, sq1, sk1, sv1, se1):
    c = lax.axis_index("c")
    s = lax.axis_index("s")

    def _zero_row(i, _):
        for j in range(ROW_W // 16):
            zbuf[i, pl.ds(j * 16, 16)] = jnp.zeros((16,), jnp.float32)
        return 0
    lax.fori_loop(0, EB, _zero_row, 0)

    def _zero_blk(t, _):
        blk = s + SC_TILES * t
        @pl.when(blk < ACC_BLKS)
        def _():
            pltpu.sync_copy(zbuf, acc.at[pl.ds(blk * EB, EB)])
        return 0
    lax.fori_loop(0, (ACC_BLKS + SC_TILES - 1) // SC_TILES, _zero_blk, 0)
    plsc.subcore_barrier()

    qoff = c * N_USERS
    soff = c * N_NODES
    eoff4 = c * (N_EDGES // 4)

    bufs = ((didx0, didx20, sidx0, q0, k0, v0, e0, sq0, sk0, sv0, se0),
            (didx1, didx21, sidx1, q1, k1, v1, e1, sq1, sk1, sv1, se1))

    def _fire(p, t):
        di, di2, si, qr, kr, vr, er, q_s, k_s, v_s, e_s = bufs[p]
        b = s + SC_TILES * t
        @pl.when(b < NBLK)
        def _():
            base = b * EB
            pltpu.sync_copy(dst_hbm.at[pl.ds(base, EB)], di)
            pltpu.sync_copy(src_hbm.at[pl.ds(base, EB)], si)
            for j in range(EB // 16):
                sl = pl.ds(j * 16, 16)
                di2[sl] = di[sl] + qoff
                si[sl] = si[sl] + soff
            pltpu.async_copy(q_hbm.at[di2], qr, q_s)
            pltpu.async_copy(k_hbm.at[si], kr, k_s)
            pltpu.async_copy(v_hbm.at[si], vr, v_s)
            pltpu.async_copy(e_hbm.at[pl.ds(eoff4 + b * EB4, EB4)], er, e_s)

    dnums = lax.GatherDimensionNumbers(
        offset_dims=(), collapsed_slice_dims=(0,), start_index_map=(0,))

    def _bcast_last(x):
        c15 = jnp.full((16, 1), 15, jnp.int32)
        return lax.gather(x, c15, dnums, (1,),
                          mode=lax.GatherScatterMode.PROMISE_IN_BOUNDS)

    def _consume(p, t):
        di, di2, si, qr, kr, vr, er, q_s, k_s, v_s, e_s = bufs[p]
        b = s + SC_TILES * t
        @pl.when(b < NBLK)
        def _():
            pltpu.make_async_copy(q_hbm.at[di2], qr, q_s).wait()
            pltpu.make_async_copy(k_hbm.at[si], kr, k_s).wait()
            pltpu.make_async_copy(v_hbm.at[si], vr, v_s).wait()
            pltpu.make_async_copy(e_hbm.at[pl.ds(0, EB4)], er, e_s).wait()

            @plsc.parallel_loop(0, EB, unroll=4)
            def _edge(i):
                iotl = lax.iota(jnp.int32, 16)
                r = i // 4
                sub = (i - 4 * r) * TBL_W
                avec = jnp.zeros((16,), jnp.float32)
                for h in range(2):
                    sl = pl.ds(h * HEAD_DIM, HEAD_DIM)
                    ev = er[r, pl.ds(sub + h * HEAD_DIM, HEAD_DIM)]
                    prod = qr[i, sl] * (kr[i, sl] + ev)
                    af = _bcast_last(jnp.cumsum(prod))
                    avec = jnp.where(iotl == h, af, avec)
                    wsp = jnp.exp(af)
                    cbuf[i, sl] = (vr[i, sl] + ev) * wsp
                cbuf[i, pl.ds(2 * HEAD_DIM, 16)] = jnp.exp(avec)
            pltpu.sync_copy(cbuf, acc.at[di], add=True)

    _fire(0, 0)

    def _step(u, _):
        _fire(1, 2 * u + 1)
        _consume(0, 2 * u)
        _fire(0, 2 * u + 2)
        _consume(1, 2 * u + 1)
        return 0
    lax.fori_loop(0, NU, _step, 0)

    plsc.subcore_barrier()

    def _drain_blk(t, _):
        blk = s + SC_TILES * t
        @pl.when(blk < ACC_BLKS)
        def _():
            pltpu.sync_copy(acc.at[pl.ds(blk * EB, EB)],
                            out_hbm.at[c, pl.ds(blk * EB, EB)])
        return 0
    lax.fori_loop(0, (ACC_BLKS + SC_TILES - 1) // SC_TILES, _drain_blk, 0)


def _sc_message_passing(q2, k2, v2, e2, src_shift, dst):
    f32 = jnp.float32
    i32 = jnp.int32
    mesh = plsc.VectorSubcoreMesh(core_axis_name="c", subcore_axis_name="s",
                                  num_cores=SC_CORES, num_subcores=SC_TILES)
    bufset = [pltpu.VMEM((EB,), i32), pltpu.VMEM((EB,), i32),
              pltpu.VMEM((EB,), i32),
              pltpu.VMEM((EB, TBL_W), f32), pltpu.VMEM((EB, TBL_W), f32),
              pltpu.VMEM((EB, TBL_W), f32), pltpu.VMEM((EB4, 128), f32)]
    run = pl.kernel(
        _sc_body,
        out_type=jax.ShapeDtypeStruct((SC_CORES, ACC_ROWS, ROW_W), f32),
        mesh=mesh,
        scratch_types=bufset + bufset + [
            pltpu.VMEM((EB, ROW_W), f32),
            pltpu.VMEM((EB, ROW_W), f32),
            pltpu.VMEM_SHARED((ACC_ROWS, ROW_W), f32),
        ] + [pltpu.SemaphoreType.DMA] * 8,
        compiler_params=pltpu.CompilerParams(use_tc_tiling_on_sc=False,
                                             needs_layout_passes=False),
    )
    return run(q2, k2, v2, e2, src_shift, dst)


def kernel(prods_only, user_features, product_info_features, product_name_features,
           edge_index, edge_attr,
           uW1, ub1, ug, ubeta, uW2, ub2,
           nW1, nb1, ng, nbeta, nW2, nb2,
           iW1, ib1, ig, ibeta, iW2, ib2,
           eW1, eb1, eg, ebeta, eW2, eb2,
           pW, pb, Wq, bq, Wk, bk, Wv, bv, We, Wskip, bskip):
    f32 = jnp.float32
    nb = N_USERS // NODE_BLK

    def _heads_out():
        return pl.BlockSpec((SC_CORES, NODE_BLK, TBL_W), lambda i: (0, i, 0))

    q2u, k2u, v2u, skip_u = pl.pallas_call(
        _user_body,
        grid=(nb,),
        in_specs=[pl.BlockSpec((NODE_BLK, D_NODE), lambda i: (i, 0)),
                  _rep(uW1.shape), _rep(ub1.shape), _rep(ug.shape), _rep(ubeta.shape),
                  _rep(uW2.shape), _rep(ub2.shape),
                  _rep(Wq.shape), _rep(bq.shape), _rep(Wk.shape), _rep(bk.shape),
                  _rep(Wv.shape), _rep(bv.shape), _rep(Wskip.shape), _rep(bskip.shape)],
        out_specs=[_heads_out(), _heads_out(), _heads_out(),
                   pl.BlockSpec((NODE_BLK, OUT_NODE), lambda i: (i, 0))],
        out_shape=[jax.ShapeDtypeStruct((SC_CORES, N_USERS, TBL_W), f32)] * 3
                  + [jax.ShapeDtypeStruct((N_USERS, OUT_NODE), f32)],
    )(user_features, uW1, ub1, ug, ubeta, uW2, ub2,
      Wq, bq, Wk, bk, Wv, bv, Wskip, bskip)

    product_emb, k2p, v2p = pl.pallas_call(
        _product_body,
        grid=(nb,),
        in_specs=[pl.BlockSpec((NODE_BLK, D_NODE), lambda i: (i, 0)),
                  pl.BlockSpec((NODE_BLK, D_NODE), lambda i: (i, 0)),
                  _rep(iW1.shape), _rep(ib1.shape), _rep(ig.shape), _rep(ibeta.shape),
                  _rep(iW2.shape), _rep(ib2.shape),
                  _rep(nW1.shape), _rep(nb1.shape), _rep(ng.shape), _rep(nbeta.shape),
                  _rep(nW2.shape), _rep(nb2.shape),
                  _rep(pW.shape), _rep(pb.shape),
                  _rep(Wk.shape), _rep(bk.shape), _rep(Wv.shape), _rep(bv.shape)],
        out_specs=[pl.BlockSpec((NODE_BLK, OUT_NODE), lambda i: (i, 0)),
                   _heads_out(), _heads_out()],
        out_shape=[jax.ShapeDtypeStruct((N_PRODUCTS, OUT_NODE), f32)]
                  + [jax.ShapeDtypeStruct((SC_CORES, N_PRODUCTS, TBL_W), f32)] * 2,
    )(product_info_features, product_name_features,
      iW1, ib1, ig, ibeta, iW2, ib2,
      nW1, nb1, ng, nbeta, nW2, nb2,
      pW, pb, Wk, bk, Wv, bv)

    dst = edge_index[1]
    maxdst = pl.pallas_call(
        _max_body,
        in_specs=[pl.BlockSpec((800, 1000), lambda: (0, 0))],
        out_specs=pl.BlockSpec((8, 128), lambda: (0, 0)),
        out_shape=jax.ShapeDtypeStruct((8, 128), jnp.int32),
    )(dst.reshape(800, 1000))
    off_blk = maxdst + 1 + jnp.asarray(prods_only, jnp.int32)

    neb = N_EDGES // EDGE_BLK
    e2, src_shift3 = pl.pallas_call(
        _edge_body,
        grid=(neb,),
        in_specs=[pl.BlockSpec((EDGE_BLK, D_EDGE), lambda i: (i, 0)),
                  pl.BlockSpec((1, 1, EDGE_BLK), lambda i: (i, 0, 0)),
                  pl.BlockSpec((8, 128), lambda i: (0, 0)),
                  _rep(eW1.shape), _rep(eb1.shape), _rep(eg.shape), _rep(ebeta.shape),
                  _rep(eW2.shape), _rep(eb2.shape), _rep(We.shape)],
        out_specs=[pl.BlockSpec((SC_CORES, EDGE_BLK // 4, 128), lambda i: (0, i, 0)),
                   pl.BlockSpec((1, 1, EDGE_BLK), lambda i: (i, 0, 0))],
        out_shape=[jax.ShapeDtypeStruct((SC_CORES, N_EDGES // 4, 128), f32),
                   jax.ShapeDtypeStruct((neb, 1, EDGE_BLK), jnp.int32)],
    )(edge_attr, edge_index[0].reshape(neb, 1, EDGE_BLK), off_blk,
      eW1, eb1, eg, ebeta, eW2, eb2, We)

    k_flat = jnp.concatenate([k2u, k2p], axis=1).reshape(SC_CORES * N_NODES, TBL_W)
    v_flat = jnp.concatenate([v2u, v2p], axis=1).reshape(SC_CORES * N_NODES, TBL_W)
    q_flat = q2u.reshape(SC_CORES * N_USERS, TBL_W)
    e_flat = e2.reshape(SC_CORES * N_EDGES // 4, 128)

    out_sc = _sc_message_passing(q_flat, k_flat, v_flat, e_flat,
                                 src_shift3.reshape(N_EDGES), dst)

    out_u = pl.pallas_call(
        _combine_body,
        grid=(nb,),
        in_specs=[pl.BlockSpec((NODE_BLK, ROW_W), lambda i: (i, 0)),
                  pl.BlockSpec((NODE_BLK, ROW_W), lambda i: (i, 0)),
                  pl.BlockSpec((NODE_BLK, OUT_NODE), lambda i: (i, 0))],
        out_specs=pl.BlockSpec((NODE_BLK, OUT_NODE), lambda i: (i, 0)),
        out_shape=jax.ShapeDtypeStruct((N_USERS, OUT_NODE), f32),
    )(out_sc[0, :N_USERS], out_sc[1, :N_USERS], skip_u)

    return (out_u, product_emb)
